# Initial kernel scaffold; baseline (speedup 1.0000x reference)
import jax
import jax.numpy as jnp
from jax.experimental import pallas as pl


def kernel(w, v, theta, rays, image_indices, depth_mask):
    raise NotImplementedError("write your pallas kernel here")


# speedup vs baseline: 14.6595x; 14.6595x over previous
import functools

import jax
import jax.numpy as jnp
from jax import lax
from jax.experimental import pallas as pl
from jax.experimental.pallas import tpu as pltpu, tpu_sc as plsc

_NC = 2
_NS = 16
_L = 16
_C = 1024


def _table_body(w_ref, v_ref, th_ref, out_ref):
    w0, w1, w2 = w_ref[0], w_ref[1], w_ref[2]
    v0, v1, v2 = v_ref[0], v_ref[1], v_ref[2]
    th = th_ref[...]
    s = jnp.sin(th)
    c1 = 1.0 - jnp.cos(th)
    tms = th - s
    a0 = w1 * v2 - w2 * v1
    a1 = w2 * v0 - w0 * v2
    a2 = w0 * v1 - w1 * v0
    b0 = w1 * a2 - w2 * a1
    b1 = w2 * a0 - w0 * a2
    b2 = w0 * a1 - w1 * a0
    out_ref[0] = w0
    out_ref[1] = w1
    out_ref[2] = w2
    out_ref[3] = s
    out_ref[4] = c1
    out_ref[5] = th * v0 + c1 * a0 + tms * b0
    out_ref[6] = th * v1 + c1 * a1 + tms * b1
    out_ref[7] = th * v2 + c1 * a2 + tms * b2
    z = jnp.zeros_like(th)
    for c in range(8, 16):
        out_ref[c] = z


def _build_table(w, v, theta):
    n = theta.shape[0]
    r = n // 128
    wt = w.T.reshape(3, r, 128)
    vt = v.T.reshape(3, r, 128)
    th = theta.reshape(r, 128)
    comps = pl.pallas_call(
        _table_body,
        out_shape=jax.ShapeDtypeStruct((16, r, 128), jnp.float32),
    )(wt, vt, th)
    return comps.reshape(16, n).T


def _make_sc_apply(B, n):
    W = _NC * _NS
    K = -(-B // (_C * W))
    mesh = plsc.VectorSubcoreMesh(
        core_axis_name="c", subcore_axis_name="s",
        num_cores=_NC, num_subcores=_NS)

    @functools.partial(
        pl.kernel,
        out_type=jax.ShapeDtypeStruct((B, 8), jnp.float32),
        mesh=mesh,
        scratch_types=[
            pltpu.VMEM((_C, 8), jnp.float32),
            pltpu.VMEM((_C,), jnp.int32),
            pltpu.VMEM((_C,), jnp.int32),
            pltpu.VMEM((_C, 16), jnp.float32),
            pltpu.SemaphoreType.DMA,
        ],
        compiler_params=pltpu.CompilerParams(
            needs_layout_passes=False, use_tc_tiling_on_sc=False),
    )
    def sc_apply(table_hbm, rays_hbm, idx_hbm, mask_hbm, out_hbm,
                 ray_v, idx_v, mask_v, rows_v, sem):
        wid = lax.axis_index("s") * _NC + lax.axis_index("c")
        iota = lax.iota(jnp.int32, _L)
        cols = [jnp.full((_L,), c, jnp.int32) for c in range(8)]

        def chunk_body(k, carry):
            start = jnp.minimum((wid * K + k) * _C, B - _C)
            pltpu.sync_copy(idx_hbm.at[pl.ds(start, _C)], idx_v)
            pltpu.sync_copy(mask_hbm.at[pl.ds(start, _C)], mask_v)
            pltpu.sync_copy(rays_hbm.at[pl.ds(start, _C)], ray_v)
            pltpu.async_copy(table_hbm.at[idx_v], rows_v, sem).wait()

            def group(g, c2):
                r16 = g * _L + iota
                ox = plsc.load_gather(ray_v, [r16, cols[0]])
                oy = plsc.load_gather(ray_v, [r16, cols[1]])
                oz = plsc.load_gather(ray_v, [r16, cols[2]])
                dx = plsc.load_gather(ray_v, [r16, cols[3]])
                dy = plsc.load_gather(ray_v, [r16, cols[4]])
                dz = plsc.load_gather(ray_v, [r16, cols[5]])
                w0 = plsc.load_gather(rows_v, [r16, cols[0]])
                w1 = plsc.load_gather(rows_v, [r16, cols[1]])
                w2 = plsc.load_gather(rows_v, [r16, cols[2]])
                s = plsc.load_gather(rows_v, [r16, cols[3]])
                c1 = plsc.load_gather(rows_v, [r16, cols[4]])
                t0 = plsc.load_gather(rows_v, [r16, cols[5]])
                t1 = plsc.load_gather(rows_v, [r16, cols[6]])
                t2 = plsc.load_gather(rows_v, [r16, cols[7]])
                mf = jnp.where(mask_v[pl.ds(g * _L, _L)] == 1, 1.0, 0.0)
                sm = s * mf
                c1m = c1 * mf
                cx = w1 * dz - w2 * dy
                cy = w2 * dx - w0 * dz
                cz = w0 * dy - w1 * dx
                ex = w1 * cz - w2 * cy
                ey = w2 * cx - w0 * cz
                ez = w0 * cy - w1 * cx
                plsc.store_scatter(ray_v, [r16, cols[0]], ox + t0 * mf)
                plsc.store_scatter(ray_v, [r16, cols[1]], oy + t1 * mf)
                plsc.store_scatter(ray_v, [r16, cols[2]], oz + t2 * mf)
                plsc.store_scatter(ray_v, [r16, cols[3]], dx + sm * cx + c1m * ex)
                plsc.store_scatter(ray_v, [r16, cols[4]], dy + sm * cy + c1m * ey)
                plsc.store_scatter(ray_v, [r16, cols[5]], dz + sm * cz + c1m * ez)
                return c2

            lax.fori_loop(0, _C // _L, group, 0)
            pltpu.sync_copy(ray_v, out_hbm.at[pl.ds(start, _C)])
            return carry

        lax.fori_loop(0, K, chunk_body, 0)

    return sc_apply


def kernel(w, v, theta, rays, image_indices, depth_mask):
    B = rays.shape[0]
    n = theta.shape[0]
    table = _build_table(w.astype(jnp.float32), v.astype(jnp.float32),
                         theta.astype(jnp.float32))
    idx = image_indices.reshape(-1).astype(jnp.int32)
    msk = depth_mask.reshape(-1).astype(jnp.int32)
    sc_apply = _make_sc_apply(B, n)
    return sc_apply(table, rays, idx, msk)
